# baseline (speedup 1.0000x reference)
import functools
import math

import jax
import jax.numpy as jnp
from jax import lax
from jax.experimental import pallas as pl
from jax.experimental.pallas import tpu as pltpu
from jax.experimental.pallas import tpu_sc as plsc

NUM_CORES = 2
NUM_SUBCORES = 16
NW = NUM_CORES * NUM_SUBCORES
LANES = 16
CHUNK = 16


def _make_kernel(B, S, D, V):
    s_per_w = S // NW
    n_groups = s_per_w // CHUNK
    scale = jnp.float32(math.sqrt(float(D)))
    d_regs = D // LANES

    mesh = plsc.VectorSubcoreMesh(
        core_axis_name="c", subcore_axis_name="s",
        num_cores=NUM_CORES, num_subcores=NUM_SUBCORES)

    @functools.partial(
        pl.kernel,
        mesh=mesh,
        out_type=jax.ShapeDtypeStruct((B * S, D), jnp.float32),
        scratch_types=[
            pltpu.VMEM((B, s_per_w), jnp.int32),
            pltpu.VMEM((B * CHUNK, D), jnp.float32),
            pltpu.VMEM((B * CHUNK, D), jnp.float32),
            pltpu.VMEM((CHUNK, D), jnp.float32),
            pltpu.VMEM((CHUNK, D), jnp.float32),
            pltpu.SemaphoreType.DMA,
            pltpu.SemaphoreType.DMA,
            pltpu.SemaphoreType.DMA,
        ],
    )
    def emb_kernel(x_hbm, wt_hbm, pos_hbm, out_hbm,
                   idx_v, wbuf0, wbuf1, pbuf0, pbuf1, gsem, ssem, psem):
        wid = lax.axis_index("s") * NUM_CORES + lax.axis_index("c")
        s_base = wid * s_per_w
        for b in range(B):
            pltpu.sync_copy(x_hbm.at[b, pl.ds(s_base, s_per_w)],
                            idx_v.at[b])

        wbufs = (wbuf0, wbuf1)
        pbufs = (pbuf0, pbuf1)

        def start_gathers(g, buf):
            for b in range(B):
                pltpu.async_copy(
                    wt_hbm.at[idx_v.at[b, pl.ds(g * CHUNK, CHUNK)]],
                    buf.at[pl.ds(b * CHUNK, CHUNK)], gsem)

        def wait_gathers(buf):
            for b in range(B):
                pltpu.make_async_copy(
                    wt_hbm.at[pl.ds(0, CHUNK)],
                    buf.at[pl.ds(b * CHUNK, CHUNK)], gsem).wait()

        def start_pos(g, buf):
            pltpu.async_copy(
                pos_hbm.at[pl.ds(s_base + g * CHUNK, CHUNK)], buf, psem)

        def wait_pos(buf):
            pltpu.make_async_copy(
                pos_hbm.at[pl.ds(0, CHUNK)], buf, psem).wait()

        def start_stores(g, buf):
            for b in range(B):
                row = b * S + s_base + g * CHUNK
                pltpu.async_copy(
                    buf.at[pl.ds(b * CHUNK, CHUNK)],
                    out_hbm.at[pl.ds(row, CHUNK)], ssem)

        def wait_stores(buf):
            for b in range(B):
                pltpu.make_async_copy(
                    buf.at[pl.ds(b * CHUNK, CHUNK)],
                    out_hbm.at[pl.ds(0, CHUNK)], ssem).wait()

        start_gathers(0, wbuf0)
        start_pos(0, pbuf0)

        def pair_body(gp, _):
            for q in range(2):
                g = gp * 2 + q
                wb = wbufs[q]
                wb_other = wbufs[1 - q]
                pb = pbufs[q]
                pb_other = pbufs[1 - q]

                @pl.when(g < n_groups - 1)
                def _():
                    @pl.when(g >= 1)
                    def _():
                        wait_stores(wb_other)
                    start_gathers(g + 1, wb_other)
                    start_pos(g + 1, pb_other)

                wait_gathers(wb)
                wait_pos(pb)

                def row_body(r, _):
                    for j in range(d_regs):
                        sl = pl.ds(j * LANES, LANES)
                        p = pb[r, sl] * scale
                        for b in range(B):
                            wb[b * CHUNK + r, sl] = wb[b * CHUNK + r, sl] * scale + p
                    return 0

                lax.fori_loop(0, CHUNK, row_body, 0)
                start_stores(g, wb)
            return 0

        lax.fori_loop(0, n_groups // 2, pair_body, 0)
        wait_stores(wbuf0)
        wait_stores(wbuf1)

    return emb_kernel


def kernel(x, word_table, pos_table):
    B, S = x.shape
    V, D = word_table.shape
    emb = _make_kernel(B, S, D, V)
    out = emb(x, word_table, pos_table[:S])
    return out.reshape(B, S, D)
